# baseline (speedup 1.0000x reference)
import functools

import numpy as np
import jax
import jax.numpy as jnp
from jax import lax
from jax.experimental import pallas as pl
from jax.experimental.pallas import tpu as pltpu
from jax.experimental.pallas import tpu_sc as plsc

NUM_LEVELS = 8
HASHMAP_SIZE = 1 << 16
RESOLUTIONS = (16, 22, 30, 41, 56, 77, 105, 144)
SIZES = [min(r ** 3, HASHMAP_SIZE) for r in RESOLUTIONS]
OFFS = np.concatenate([[0], np.cumsum(SIZES)]).astype(np.int64)
N_TOTAL = int(OFFS[-1])
K3 = 27
CIN = 8
COUT = 8
PRIMES = (1, 2654435761, 805459861)

CHUNK = 64
NW = 32
GROUPS = CHUNK // 16


def _neighbor_table(level):
    R = RESOLUTIONS[level]
    size = SIZES[level]
    idx = np.arange(size, dtype=np.int64)
    x = idx % R
    y = (idx // R) % R
    z = (idx // (R * R)) % R
    hashed = R ** 3 > HASHMAP_SIZE
    cols = []
    for dz in (-1, 0, 1):
        for dy in (-1, 0, 1):
            for dx in (-1, 0, 1):
                nx = np.clip(x + dx, 0, R - 1)
                ny = np.clip(y + dy, 0, R - 1)
                nz = np.clip(z + dz, 0, R - 1)
                if hashed:
                    h = (nx * PRIMES[0]) ^ (ny * PRIMES[1]) ^ (nz * PRIMES[2])
                    nidx = h % HASHMAP_SIZE
                else:
                    nidx = nx + ny * R + nz * R * R
                cols.append(nidx + OFFS[level])
    return np.stack(cols, axis=1).astype(np.int32)


def _build_chunks():
    nch_per_level = [-(-s // CHUNK) for s in SIZES]
    nch = sum(nch_per_level)
    nit = -(-nch // NW)
    nit += nit % 2
    nchp = nit * NW
    cs = [0]
    for v in nch_per_level:
        cs.append(cs[-1] + v)
    neigh_pad = np.zeros((nchp * CHUNK, K3), np.int32)
    for lvl in range(NUM_LEVELS):
        base = cs[lvl] * CHUNK
        neigh_pad[base:base + SIZES[lvl]] = _neighbor_table(lvl)
    nidx = np.ascontiguousarray(
        neigh_pad.reshape(nchp, CHUNK, K3).transpose(0, 2, 1)
    ).reshape(nchp, K3 * CHUNK)
    nidx = np.concatenate([nidx, np.zeros((1, K3 * CHUNK), np.int32)])
    return nidx, cs, nit, nchp


_NIDX_NP, _CS, NIT, NCHP = _build_chunks()
NPAD = NCHP * CHUNK


def _lvl_of(c):
    lvl = jnp.int32(0)
    for s in _CS[1:NUM_LEVELS]:
        lvl = lvl + (c >= s).astype(jnp.int32)
    return lvl


def _conv_body(table, wrep, brep, nidx, out, idxbuf, gbuf, wbuf, bbuf, obuf,
               sem0, sem1):
    wid = lax.axis_index("c") * 16 + lax.axis_index("s")
    iota = lax.iota(jnp.int32, 16)
    NROWS = K3 * CHUNK
    sems = (sem0, sem1)

    def start_gather(c, p):
        pltpu.sync_copy(nidx.at[c], idxbuf.at[p])
        return pltpu.async_copy(
            table.at[idxbuf.at[p]],
            gbuf.at[pl.ds(p * NROWS, NROWS)], sems[p])

    def compute_chunk(c, k, p):
        lvl = _lvl_of(c)

        @pl.when((k == 0) | (lvl != _lvl_of(c - 1)))
        def _load_weights():
            pltpu.sync_copy(wrep.at[lvl], wbuf)
            pltpu.sync_copy(brep.at[lvl], bbuf)

        pltpu.make_async_copy(
            table.at[idxbuf.at[p]],
            gbuf.at[pl.ds(p * NROWS, NROWS)], sems[p]).wait()

        pbase = p * NROWS
        bias_v = [bbuf[pl.ds(o * 16, 16)] for o in range(COUT)]
        for g in range(GROUPS):
            rv = iota + (g * 16 + pbase)
            acc0 = tuple(bias_v) + tuple(bias_v)

            def tap_body(t, accs):
                accs = list(accs)
                frv = rv + t * CHUNK
                wbase = t * (CIN * COUT)
                wv = [wbuf[pl.ds(wbase + q * 16, 16)] for q in range(4)]
                for i in range(CIN):
                    gv = [
                        plsc.load_gather(
                            gbuf,
                            [frv, jnp.full((16,), b * 8 + i, jnp.int32)])
                        for b in range(2)
                    ]
                    for o in range(COUT):
                        j = i * COUT + o
                        w = lax.gather(
                            wv[j // 16],
                            jnp.full((16, 1), j % 16, jnp.int32),
                            lax.GatherDimensionNumbers(
                                offset_dims=(),
                                collapsed_slice_dims=(0,),
                                start_index_map=(0,)),
                            (1,),
                            mode=lax.GatherScatterMode.PROMISE_IN_BOUNDS)
                        accs[o] = accs[o] + gv[0] * w
                        accs[8 + o] = accs[8 + o] + gv[1] * w
                return tuple(accs)

            accs = plsc.parallel_loop(0, K3, unroll=1, carry=acc0)(tap_body)
            for b in range(2):
                for o in range(COUT):
                    obuf[pl.ds((b * 8 + o) * CHUNK + g * 16, 16)] = (
                        accs[b * 8 + o])

        pltpu.sync_copy(obuf, out.at[pl.ds(c * (CHUNK * 16), CHUNK * 16)])

    start_gather(wid * NIT, 0)

    def pair_body(h, carry):
        k0 = h * 2
        c0 = wid * NIT + k0
        start_gather(c0 + 1, 1)
        compute_chunk(c0, k0, 0)
        start_gather(c0 + 2, 0)
        compute_chunk(c0 + 1, k0 + 1, 1)
        return carry

    lax.fori_loop(0, NIT // 2, pair_body, 0)
    pltpu.make_async_copy(
        table.at[idxbuf.at[0]],
        gbuf.at[pl.ds(0, NROWS)], sems[0]).wait()


def kernel(input, weight, bias):
    table = jnp.transpose(input, (1, 0, 2)).reshape(N_TOTAL, 16)
    wrep = weight.reshape(NUM_LEVELS, K3 * CIN * COUT)
    brep = jnp.broadcast_to(bias[:, :, None], (NUM_LEVELS, COUT, 16))
    brep = brep.reshape(NUM_LEVELS, COUT * 16)
    nidx = jnp.asarray(_NIDX_NP)

    mesh = plsc.VectorSubcoreMesh(core_axis_name="c", subcore_axis_name="s")
    fn = pl.kernel(
        _conv_body,
        out_type=jax.ShapeDtypeStruct((NPAD * 16,), jnp.float32),
        mesh=mesh,
        compiler_params=pltpu.CompilerParams(
            needs_layout_passes=False, use_tc_tiling_on_sc=False),
        scratch_types=[
            pltpu.VMEM((2, K3 * CHUNK), jnp.int32),
            pltpu.VMEM((2 * K3 * CHUNK, 16), jnp.float32),
            pltpu.VMEM((K3 * CIN * COUT,), jnp.float32),
            pltpu.VMEM((COUT * 16,), jnp.float32),
            pltpu.VMEM((CHUNK * 16,), jnp.float32),
            pltpu.SemaphoreType.DMA,
            pltpu.SemaphoreType.DMA,
        ],
    )
    out_pad = fn(table, wrep, brep, nidx).reshape(NCHP, 16, CHUNK)
    out_pad = jnp.transpose(out_pad, (0, 2, 1)).reshape(NPAD, 16)
    parts = [
        out_pad[_CS[l] * CHUNK:_CS[l] * CHUNK + SIZES[l]]
        for l in range(NUM_LEVELS)
    ]
    out = jnp.concatenate(parts, axis=0).reshape(N_TOTAL, 2, COUT)
    return jnp.transpose(out, (1, 0, 2))


# speedup vs baseline: 1.2229x; 1.2229x over previous
import functools

import numpy as np
import jax
import jax.numpy as jnp
from jax import lax
from jax.experimental import pallas as pl
from jax.experimental.pallas import tpu as pltpu
from jax.experimental.pallas import tpu_sc as plsc

NUM_LEVELS = 8
HASHMAP_SIZE = 1 << 16
RESOLUTIONS = (16, 22, 30, 41, 56, 77, 105, 144)
SIZES = [min(r ** 3, HASHMAP_SIZE) for r in RESOLUTIONS]
OFFS = np.concatenate([[0], np.cumsum(SIZES)]).astype(np.int64)
N_TOTAL = int(OFFS[-1])
K3 = 27
CIN = 8
COUT = 8
PRIMES = (1, 2654435761, 805459861)

CHUNK = 64
NW = 32
GROUPS = CHUNK // 16


def _neighbor_table(level):
    R = RESOLUTIONS[level]
    size = SIZES[level]
    idx = np.arange(size, dtype=np.int64)
    x = idx % R
    y = (idx // R) % R
    z = (idx // (R * R)) % R
    hashed = R ** 3 > HASHMAP_SIZE
    cols = []
    for dz in (-1, 0, 1):
        for dy in (-1, 0, 1):
            for dx in (-1, 0, 1):
                nx = np.clip(x + dx, 0, R - 1)
                ny = np.clip(y + dy, 0, R - 1)
                nz = np.clip(z + dz, 0, R - 1)
                if hashed:
                    h = (nx * PRIMES[0]) ^ (ny * PRIMES[1]) ^ (nz * PRIMES[2])
                    nidx = h % HASHMAP_SIZE
                else:
                    nidx = nx + ny * R + nz * R * R
                cols.append(nidx + OFFS[level])
    return np.stack(cols, axis=1).astype(np.int32)


def _build_chunks():
    nch_per_level = [-(-s // CHUNK) for s in SIZES]
    nch = sum(nch_per_level)
    nit = -(-nch // NW)
    nit += nit % 2
    nchp = nit * NW
    cs = [0]
    for v in nch_per_level:
        cs.append(cs[-1] + v)
    neigh_pad = np.zeros((nchp * CHUNK, K3), np.int32)
    for lvl in range(NUM_LEVELS):
        base = cs[lvl] * CHUNK
        neigh_pad[base:base + SIZES[lvl]] = _neighbor_table(lvl)
    nidx = np.ascontiguousarray(
        neigh_pad.reshape(nchp, CHUNK, K3).transpose(0, 2, 1)
    ).reshape(nchp, K3 * CHUNK)
    nidx = np.concatenate([nidx, np.zeros((1, K3 * CHUNK), np.int32)])
    return nidx, cs, nit, nchp


_NIDX_NP, _CS, NIT, NCHP = _build_chunks()
NPAD = NCHP * CHUNK


def _lvl_of(c):
    lvl = jnp.int32(0)
    for s in _CS[1:NUM_LEVELS]:
        lvl = lvl + (c >= s).astype(jnp.int32)
    return lvl


def _conv_body(table, wrep, brep, nidx, out, idxbuf, gbuf, wbuf, bbuf, obuf,
               sem0, sem1):
    wid = lax.axis_index("c") * 16 + lax.axis_index("s")
    iota = lax.iota(jnp.int32, 16)
    NROWS = K3 * CHUNK
    sems = (sem0, sem1)

    def start_gather(c, p):
        pltpu.sync_copy(nidx.at[c], idxbuf.at[p])
        return pltpu.async_copy(
            table.at[idxbuf.at[p]],
            gbuf.at[pl.ds(p * NROWS, NROWS)], sems[p])

    def compute_chunk(c, k, p):
        lvl = _lvl_of(c)

        @pl.when((k == 0) | (lvl != _lvl_of(c - 1)))
        def _load_weights():
            pltpu.sync_copy(wrep.at[lvl], wbuf)
            pltpu.sync_copy(brep.at[lvl], bbuf)

        pltpu.make_async_copy(
            table.at[idxbuf.at[p]],
            gbuf.at[pl.ds(p * NROWS, NROWS)], sems[p]).wait()

        pbase = p * NROWS
        bias_v = [bbuf[pl.ds(o * 16, 16)] for o in range(COUT)]
        for g in range(GROUPS):
            rv = iota + (g * 16 + pbase)
            acc0 = tuple(bias_v) + tuple(bias_v)

            def tap_body(t, accs):
                accs = list(accs)
                frv = rv + t * CHUNK
                wbase = t * (CIN * COUT * 16)
                for i in range(CIN):
                    gv = [
                        plsc.load_gather(
                            gbuf,
                            [frv, jnp.full((16,), b * 8 + i, jnp.int32)])
                        for b in range(2)
                    ]
                    for o in range(COUT):
                        w = wbuf[pl.ds(wbase + (i * COUT + o) * 16, 16)]
                        accs[o] = accs[o] + gv[0] * w
                        accs[8 + o] = accs[8 + o] + gv[1] * w
                return tuple(accs)

            accs = plsc.parallel_loop(0, K3, unroll=1, carry=acc0)(tap_body)
            for b in range(2):
                for o in range(COUT):
                    obuf[pl.ds((b * 8 + o) * CHUNK + g * 16, 16)] = (
                        accs[b * 8 + o])

        pltpu.sync_copy(obuf, out.at[pl.ds(c * (CHUNK * 16), CHUNK * 16)])

    start_gather(wid * NIT, 0)

    def pair_body(h, carry):
        k0 = h * 2
        c0 = wid * NIT + k0
        start_gather(c0 + 1, 1)
        compute_chunk(c0, k0, 0)
        start_gather(c0 + 2, 0)
        compute_chunk(c0 + 1, k0 + 1, 1)
        return carry

    lax.fori_loop(0, NIT // 2, pair_body, 0)
    pltpu.make_async_copy(
        table.at[idxbuf.at[0]],
        gbuf.at[pl.ds(0, NROWS)], sems[0]).wait()


def kernel(input, weight, bias):
    table = jnp.transpose(input, (1, 0, 2)).reshape(N_TOTAL, 16)
    w2 = weight.reshape(NUM_LEVELS, K3 * CIN * COUT)
    wrep = jnp.broadcast_to(w2[:, :, None],
                            (NUM_LEVELS, K3 * CIN * COUT, 16))
    wrep = wrep.reshape(NUM_LEVELS, K3 * CIN * COUT * 16)
    brep = jnp.broadcast_to(bias[:, :, None], (NUM_LEVELS, COUT, 16))
    brep = brep.reshape(NUM_LEVELS, COUT * 16)
    nidx = jnp.asarray(_NIDX_NP)

    mesh = plsc.VectorSubcoreMesh(core_axis_name="c", subcore_axis_name="s")
    fn = pl.kernel(
        _conv_body,
        out_type=jax.ShapeDtypeStruct((NPAD * 16,), jnp.float32),
        mesh=mesh,
        compiler_params=pltpu.CompilerParams(
            needs_layout_passes=False, use_tc_tiling_on_sc=False),
        scratch_types=[
            pltpu.VMEM((2, K3 * CHUNK), jnp.int32),
            pltpu.VMEM((2 * K3 * CHUNK, 16), jnp.float32),
            pltpu.VMEM((K3 * CIN * COUT * 16,), jnp.float32),
            pltpu.VMEM((COUT * 16,), jnp.float32),
            pltpu.VMEM((CHUNK * 16,), jnp.float32),
            pltpu.SemaphoreType.DMA,
            pltpu.SemaphoreType.DMA,
        ],
    )
    out_pad = fn(table, wrep, brep, nidx).reshape(NCHP, 16, CHUNK)
    out_pad = jnp.transpose(out_pad, (0, 2, 1)).reshape(NPAD, 16)
    parts = [
        out_pad[_CS[l] * CHUNK:_CS[l] * CHUNK + SIZES[l]]
        for l in range(NUM_LEVELS)
    ]
    out = jnp.concatenate(parts, axis=0).reshape(N_TOTAL, 2, COUT)
    return jnp.transpose(out, (1, 0, 2))
